# Initial kernel scaffold; baseline (speedup 1.0000x reference)
import jax
import jax.numpy as jnp
from jax.experimental import pallas as pl


def kernel(embeddings, token_type_ids, attention_mask, W, b):
    raise NotImplementedError("write your pallas kernel here")


# speedup vs baseline: 8.6927x; 8.6927x over previous
import jax
import jax.numpy as jnp
from jax import lax
from jax.experimental import pallas as pl

_SEQ = 512
_ANS = 8
_B = 4
_C = 384
_P = 4068


def _body(emb_ref, tt_ref, am_ref, w_ref, b_ref, out_ref):
    emb = emb_ref[...]
    w = w_ref[...]
    mask = tt_ref[...] * am_ref[...]
    bias = b_ref[0, 0]

    S = jnp.sum(emb[:, :, :_C] * w[None, :, :], axis=-1)
    E = jnp.sum(emb[:, :, _C:] * w[None, :, :], axis=-1)

    lane = lax.broadcasted_iota(jnp.int32, (1, _SEQ), 1)
    zs = jnp.zeros((_B, 1), dtype=jnp.float32)
    zm = jnp.zeros((1, 1), dtype=jnp.float32)

    grids = []
    for d in range(_ANS):
        if d == 0:
            E_sh, M_sh = E, mask
        else:
            E_sh = jnp.concatenate([E[:, d:], jnp.tile(zs, (1, d))], axis=1)
            M_sh = jnp.concatenate([mask[:, d:], jnp.tile(zm, (1, d))], axis=1)
        L = (jnp.broadcast_to(S[None, :, :], (_B, _B, _SEQ))
             + jnp.broadcast_to(E_sh[:, None, :], (_B, _B, _SEQ))
             ).reshape(_B * _B, _SEQ)
        L = L + bias - 1e7 * (1.0 - mask * M_sh)
        L = jnp.where(lane < _SEQ - d, L, -1e30)
        grids.append(L)

    G = jnp.stack(grids, axis=-1).reshape(_B * _B, _SEQ * _ANS)
    m = jnp.max(G, axis=-1, keepdims=True)
    ex = jnp.exp(G - m)
    s = jnp.sum(ex, axis=-1, keepdims=True)
    P = ex / s

    pieces = [P[:, : 505 * _ANS]]
    for k in range(7):
        gs = (505 + k) * _ANS
        pieces.append(P[:, gs : gs + 7 - k])
    out_ref[...] = jnp.concatenate(pieces, axis=-1)


def kernel(embeddings, token_type_ids, attention_mask, W, b):
    ttf = token_type_ids.astype(jnp.float32).reshape(1, _SEQ)
    amf = attention_mask.astype(jnp.float32).reshape(1, _SEQ)
    wr = W.reshape(1, _C)
    br = b.reshape(1, 1)
    out = pl.pallas_call(
        _body,
        out_shape=jax.ShapeDtypeStruct((_B * _B, _P), jnp.float32),
    )(embeddings, ttf, amf, wr, br)
    return out.reshape(_B, _B, _P)
